# Initial kernel scaffold; baseline (speedup 1.0000x reference)
import jax
import jax.numpy as jnp
from jax.experimental import pallas as pl


def kernel(x, edge_index, W1_l, b1_l, W1_r, W2_l, b2_l, W2_r, W3_l, b3_l, W3_r):
    raise NotImplementedError("write your pallas kernel here")


# speedup vs baseline: 7.1625x; 7.1625x over previous
import functools

import jax
import jax.numpy as jnp
from jax import lax
from jax.experimental import pallas as pl
from jax.experimental.pallas import tpu as pltpu
from jax.experimental.pallas import tpu_sc as plsc

N = 10000
E = 320000
NPAD = 10240
D_IN = 128

NTILE = 16
NCORE = 2
CHUNK = 128
NCHUNK = 160
GRP = 8
NGRP = NCHUNK // GRP
EPT = CHUNK * NCHUNK
EPAD = EPT * NTILE
ROWS_PT = NPAD // NTILE


def _make_agg(W, with_counts):
    mesh = plsc.VectorSubcoreMesh(core_axis_name="c", subcore_axis_name="s")
    out_type = [jax.ShapeDtypeStruct((2 * NPAD, W), jnp.float32)]
    scratch = [
        pltpu.VMEM_SHARED((NPAD, W), jnp.float32),
        pltpu.VMEM((GRP, CHUNK), jnp.int32),
        pltpu.VMEM((GRP, CHUNK), jnp.int32),
        pltpu.VMEM((CHUNK, W), jnp.float32),
        pltpu.SemaphoreType.DMA,
    ]
    if with_counts:
        out_type.append(jax.ShapeDtypeStruct((NPAD, 8), jnp.float32))
        scratch += [
            pltpu.VMEM_SHARED((NPAD, 8), jnp.float32),
            pltpu.VMEM((CHUNK, 8), jnp.float32),
        ]

    def body_common(table, src2, dstt, zeros_w, out, acc, sidx, didx, rows,
                    sem, cnt_stuff):
        c = lax.axis_index("c")
        t = lax.axis_index("s")
        r0 = t * ROWS_PT
        pltpu.sync_copy(zeros_w.at[pl.ds(r0, ROWS_PT)],
                        acc.at[pl.ds(r0, ROWS_PT)])
        if cnt_stuff is not None:
            ones8, zeros8, cnt_out, cntacc, onesb = cnt_stuff
            pltpu.sync_copy(zeros8.at[pl.ds(r0, ROWS_PT)],
                            cntacc.at[pl.ds(r0, ROWS_PT)])
            pltpu.sync_copy(ones8, onesb)
        plsc.subcore_barrier()

        def group(g, carry):
            pltpu.sync_copy(src2.at[c, t, pl.ds(g * GRP, GRP)], sidx)
            pltpu.sync_copy(dstt.at[t, pl.ds(g * GRP, GRP)], didx)

            def step(j, carry2):
                pltpu.async_copy(table.at[sidx.at[j]], rows, sem).wait()
                pltpu.sync_copy(rows, acc.at[didx.at[j]], add=True)
                if cnt_stuff is not None:
                    pltpu.sync_copy(onesb, cntacc.at[didx.at[j]], add=True)
                return carry2

            return lax.fori_loop(0, GRP, step, carry)

        lax.fori_loop(0, NGRP, group, 0)
        plsc.subcore_barrier()
        pltpu.sync_copy(acc.at[pl.ds(r0, ROWS_PT)],
                        out.at[pl.ds(c * NPAD + r0, ROWS_PT)])
        if cnt_stuff is not None:
            @pl.when(c == 0)
            def _():
                pltpu.sync_copy(cntacc.at[pl.ds(r0, ROWS_PT)],
                                cnt_out.at[pl.ds(r0, ROWS_PT)])

    if with_counts:
        def body(table, src2, dstt, zeros_w, ones8, zeros8, out, cnt_out,
                 acc, sidx, didx, rows, sem, cntacc, onesb):
            body_common(table, src2, dstt, zeros_w, out, acc, sidx, didx,
                        rows, sem, (ones8, zeros8, cnt_out, cntacc, onesb))
    else:
        def body(table, src2, dstt, zeros_w, out,
                 acc, sidx, didx, rows, sem):
            body_common(table, src2, dstt, zeros_w, out, acc, sidx, didx,
                        rows, sem, None)

    out_type = tuple(out_type) if with_counts else out_type[0]
    return pl.kernel(body, out_type=out_type, mesh=mesh,
                     scratch_types=scratch,
                     compiler_params=pltpu.CompilerParams(
                         use_tc_tiling_on_sc=False))


_DOT = functools.partial(
    lax.dot_general,
    dimension_numbers=(((1,), (1,)), ((), ())),
    preferred_element_type=jnp.float32,
)

_BR = 128


def _tc_layer1(acc, cnt, x, wl, bl, wr):
    def body(acc_ref, cnt_ref, x_ref, wl_ref, bl_ref, wr_ref, h1_ref):
        agg = jnp.concatenate([acc_ref[0], acc_ref[1]], axis=1)
        agg = agg / jnp.maximum(cnt_ref[:, 0:1], 1.0)
        h = _DOT(agg, wl_ref[...]) + bl_ref[...] + _DOT(x_ref[...], wr_ref[...])
        h = jnp.maximum(h, 0.0)
        h1_ref[0] = h[:, :128]
        h1_ref[1] = h[:, 128:]

    return pl.pallas_call(
        body,
        grid=(NPAD // _BR,),
        in_specs=[
            pl.BlockSpec((2, _BR, 64), lambda i: (0, i, 0)),
            pl.BlockSpec((_BR, 8), lambda i: (i, 0)),
            pl.BlockSpec((_BR, 128), lambda i: (i, 0)),
            pl.BlockSpec((256, 128), lambda i: (0, 0)),
            pl.BlockSpec((1, 256), lambda i: (0, 0)),
            pl.BlockSpec((256, 128), lambda i: (0, 0)),
        ],
        out_specs=pl.BlockSpec((2, _BR, 128), lambda i: (0, i, 0)),
        out_shape=jax.ShapeDtypeStruct((2, NPAD, 128), jnp.float32),
    )(acc, cnt, x, wl, bl, wr)


def _tc_layer2(acc, cnt, h1, wl, bl, wr, w3l):
    def body(acc_ref, cnt_ref, h1_ref, wl_ref, bl_ref, wr_ref, w3_ref,
             h2_ref, y3_ref):
        agg = jnp.concatenate([acc_ref[0], acc_ref[1]], axis=1)
        agg = agg / jnp.maximum(cnt_ref[:, 0:1], 1.0)
        h1f = jnp.concatenate([h1_ref[0], h1_ref[1]], axis=1)
        h = _DOT(agg, wl_ref[...]) + bl_ref[...] + _DOT(h1f, wr_ref[...])
        h = jnp.maximum(h, 0.0)
        h2_ref[...] = h
        y3_ref[0] = _DOT(h, w3_ref[0])
        y3_ref[1] = _DOT(h, w3_ref[1])

    return pl.pallas_call(
        body,
        grid=(NPAD // _BR,),
        in_specs=[
            pl.BlockSpec((2, _BR, 128), lambda i: (0, i, 0)),
            pl.BlockSpec((_BR, 8), lambda i: (i, 0)),
            pl.BlockSpec((2, _BR, 128), lambda i: (0, i, 0)),
            pl.BlockSpec((512, 256), lambda i: (0, 0)),
            pl.BlockSpec((1, 512), lambda i: (0, 0)),
            pl.BlockSpec((512, 256), lambda i: (0, 0)),
            pl.BlockSpec((2, 64, 512), lambda i: (0, 0, 0)),
        ],
        out_specs=[
            pl.BlockSpec((_BR, 512), lambda i: (i, 0)),
            pl.BlockSpec((2, _BR, 64), lambda i: (0, i, 0)),
        ],
        out_shape=[
            jax.ShapeDtypeStruct((NPAD, 512), jnp.float32),
            jax.ShapeDtypeStruct((2, NPAD, 64), jnp.float32),
        ],
    )(acc, cnt, h1, wl, bl, wr, w3l)


def _tc_layer3(acc, cnt, h2, wr, bl):
    def body(acc_ref, cnt_ref, h2_ref, wr_ref, bl_ref, out_ref):
        agg = jnp.concatenate([acc_ref[0], acc_ref[1]], axis=1)
        agg = agg / jnp.maximum(cnt_ref[:, 0:1], 1.0)
        out_ref[...] = agg + bl_ref[...] + _DOT(h2_ref[...], wr_ref[...])

    return pl.pallas_call(
        body,
        grid=(NPAD // _BR,),
        in_specs=[
            pl.BlockSpec((2, _BR, 64), lambda i: (0, i, 0)),
            pl.BlockSpec((_BR, 8), lambda i: (i, 0)),
            pl.BlockSpec((_BR, 512), lambda i: (i, 0)),
            pl.BlockSpec((128, 512), lambda i: (0, 0)),
            pl.BlockSpec((1, 128), lambda i: (0, 0)),
        ],
        out_specs=pl.BlockSpec((_BR, 128), lambda i: (i, 0)),
        out_shape=jax.ShapeDtypeStruct((NPAD, 128), jnp.float32),
    )(acc, cnt, h2, wr, bl)


def kernel(x, edge_index, W1_l, b1_l, W1_r, W2_l, b2_l, W2_r, W3_l, b3_l, W3_r):
    x = x.astype(jnp.float32)
    src = edge_index[0].astype(jnp.int32)
    dst = edge_index[1].astype(jnp.int32)

    npad_e = EPAD - E
    pad_ar = jnp.arange(npad_e, dtype=jnp.int32)
    src_pad = jnp.concatenate([src, pad_ar % N])
    dst_pad = jnp.concatenate([dst, N + pad_ar % (NPAD - N)])
    src_t = src_pad.reshape(NTILE, NCHUNK, CHUNK)
    src2 = jnp.stack([src_t, src_t + NPAD])
    dst_t = dst_pad.reshape(NTILE, NCHUNK, CHUNK)

    zeros64 = jnp.zeros((NPAD, 64), jnp.float32)
    zeros128 = jnp.zeros((NPAD, 128), jnp.float32)
    zeros8 = jnp.zeros((NPAD, 8), jnp.float32)
    ones8 = jnp.ones((CHUNK, 8), jnp.float32)

    xpad = jnp.zeros((NPAD, D_IN), jnp.float32).at[:N].set(x)
    t1 = jnp.concatenate([xpad[:, :64], xpad[:, 64:]], axis=0)

    acc1, cnt = _make_agg(64, True)(t1, src2, dst_t, zeros64, ones8, zeros8)
    h1 = _tc_layer1(acc1.reshape(2, NPAD, 64), cnt, xpad,
                    W1_l, b1_l.reshape(1, -1), W1_r)

    acc2 = _make_agg(128, False)(h1.reshape(2 * NPAD, 128), src2, dst_t,
                                 zeros128)
    h2, y3 = _tc_layer2(acc2.reshape(2, NPAD, 128), cnt, h1,
                        W2_l, b2_l.reshape(1, -1), W2_r,
                        W3_l.reshape(2, 64, 512))

    acc3 = _make_agg(64, False)(y3.reshape(2 * NPAD, 64), src2, dst_t,
                                zeros64)
    out = _tc_layer3(acc3.reshape(2, NPAD, 64), cnt, h2,
                     W3_r, b3_l.reshape(1, -1))
    return out[:N]


# speedup vs baseline: 10.6248x; 1.4834x over previous
import functools

import jax
import jax.numpy as jnp
from jax import lax
from jax.experimental import pallas as pl
from jax.experimental.pallas import tpu as pltpu
from jax.experimental.pallas import tpu_sc as plsc

N = 10000
E = 320000
NPAD = 10240
D_IN = 128

NTILE = 16
NCORE = 2
CHUNK = 128
NCHUNK = 160
GRP = 8
NGRP = NCHUNK // GRP
EPT = CHUNK * NCHUNK
EPAD = EPT * NTILE
ROWS_PT = NPAD // NTILE


def _make_agg(W, with_counts):
    mesh = plsc.VectorSubcoreMesh(core_axis_name="c", subcore_axis_name="s")
    nbuf = 2 if W == 128 else 4
    npre = nbuf - 1
    out_type = [jax.ShapeDtypeStruct((2 * NPAD, W), jnp.float32)]
    scratch = [
        pltpu.VMEM_SHARED((NPAD, W), jnp.float32),
        pltpu.VMEM((2, GRP, CHUNK), jnp.int32),
        pltpu.VMEM((2, GRP, CHUNK), jnp.int32),
        [pltpu.VMEM((CHUNK, W), jnp.float32) for _ in range(nbuf)],
        [pltpu.SemaphoreType.DMA for _ in range(nbuf)],
        pltpu.SemaphoreType.DMA,
    ]
    if with_counts:
        out_type.append(jax.ShapeDtypeStruct((NPAD, 8), jnp.float32))
        scratch += [
            pltpu.VMEM_SHARED((NPAD, 8), jnp.float32),
            pltpu.VMEM((CHUNK, 8), jnp.float32),
        ]

    def body_common(table, src2, dstt, zeros_w, out, acc, sidx, didx, bufs,
                    sems, isem, c, t, cnt_stuff):
        r0 = t * ROWS_PT
        pltpu.sync_copy(zeros_w.at[pl.ds(r0, ROWS_PT)],
                        acc.at[pl.ds(r0, ROWS_PT)])
        if cnt_stuff is not None:
            ones8, zeros8, cnt_out, cntacc, onesb = cnt_stuff
            pltpu.sync_copy(zeros8.at[pl.ds(r0, ROWS_PT)],
                            cntacc.at[pl.ds(r0, ROWS_PT)])
            pltpu.sync_copy(ones8, onesb)

        def gather_start(slot, j, b):
            pltpu.async_copy(table.at[sidx.at[slot, j]], bufs[b], sems[b])

        def gather_wait(b):
            pltpu.make_async_copy(table.at[sidx.at[0, 0]], bufs[b],
                                  sems[b]).wait()

        def idx_start(g):
            off = jnp.minimum(g, NGRP - 1) * GRP
            slot = lax.rem(g, 2)
            pltpu.async_copy(src2.at[c, t, pl.ds(off, GRP)],
                             sidx.at[slot], isem)
            pltpu.async_copy(dstt.at[t, pl.ds(off, GRP)],
                             didx.at[slot], isem)

        def idx_wait(g):
            slot = lax.rem(g, 2)
            pltpu.make_async_copy(src2.at[c, t, pl.ds(0, GRP)],
                                  sidx.at[slot], isem).wait()
            pltpu.make_async_copy(dstt.at[t, pl.ds(0, GRP)],
                                  didx.at[slot], isem).wait()

        idx_start(0)
        idx_wait(0)
        for j in range(npre):
            gather_start(0, j, j)
        plsc.subcore_barrier()

        def group(g, carry):
            slot = lax.rem(g, 2)
            idx_start(g + 1)
            for j in range(GRP):
                b = j % nbuf
                gather_wait(b)
                nj = j + npre
                if nj < GRP:
                    gather_start(slot, nj, nj % nbuf)
                pltpu.sync_copy(bufs[b], acc.at[didx.at[slot, j]], add=True)
                if cnt_stuff is not None:
                    pltpu.sync_copy(onesb, cntacc.at[didx.at[slot, j]],
                                    add=True)
            idx_wait(g + 1)
            for j in range(npre):
                gather_start(lax.rem(g + 1, 2), j, j % nbuf)
            return carry

        lax.fori_loop(0, NGRP, group, 0)
        for j in range(npre):
            gather_wait(j % nbuf)
        plsc.subcore_barrier()
        pltpu.sync_copy(acc.at[pl.ds(r0, ROWS_PT)],
                        out.at[pl.ds(c * NPAD + r0, ROWS_PT)])
        if cnt_stuff is not None:
            @pl.when(c == 0)
            def _():
                pltpu.sync_copy(cntacc.at[pl.ds(r0, ROWS_PT)],
                                cnt_out.at[pl.ds(r0, ROWS_PT)])

    if with_counts:
        def body(table, src2, dstt, zeros_w, ones8, zeros8, out, cnt_out,
                 acc, sidx, didx, bufs, sems, isem, cntacc, onesb):
            c = lax.axis_index("c")
            t = lax.axis_index("s")
            body_common(table, src2, dstt, zeros_w, out, acc, sidx, didx,
                        bufs, sems, isem, c, t,
                        (ones8, zeros8, cnt_out, cntacc, onesb))
    else:
        def body(table, src2, dstt, zeros_w, out,
                 acc, sidx, didx, bufs, sems, isem):
            c = lax.axis_index("c")
            t = lax.axis_index("s")
            body_common(table, src2, dstt, zeros_w, out, acc, sidx, didx,
                        bufs, sems, isem, c, t, None)

    out_type = tuple(out_type) if with_counts else out_type[0]
    return pl.kernel(body, out_type=out_type, mesh=mesh,
                     scratch_types=scratch,
                     compiler_params=pltpu.CompilerParams(
                         use_tc_tiling_on_sc=False))


_DOT = functools.partial(
    lax.dot_general,
    dimension_numbers=(((1,), (1,)), ((), ())),
    preferred_element_type=jnp.float32,
)

_BR = 128


def _tc_layer1(acc, cnt, x, wl, bl, wr):
    def body(acc_ref, cnt_ref, x_ref, wl_ref, bl_ref, wr_ref, h1_ref):
        agg = jnp.concatenate([acc_ref[0], acc_ref[1]], axis=1)
        agg = agg / jnp.maximum(cnt_ref[:, 0:1], 1.0)
        h = _DOT(agg, wl_ref[...]) + bl_ref[...] + _DOT(x_ref[...], wr_ref[...])
        h = jnp.maximum(h, 0.0)
        h1_ref[0] = h[:, :128]
        h1_ref[1] = h[:, 128:]

    return pl.pallas_call(
        body,
        grid=(NPAD // _BR,),
        in_specs=[
            pl.BlockSpec((2, _BR, 64), lambda i: (0, i, 0)),
            pl.BlockSpec((_BR, 8), lambda i: (i, 0)),
            pl.BlockSpec((_BR, 128), lambda i: (i, 0)),
            pl.BlockSpec((256, 128), lambda i: (0, 0)),
            pl.BlockSpec((1, 256), lambda i: (0, 0)),
            pl.BlockSpec((256, 128), lambda i: (0, 0)),
        ],
        out_specs=pl.BlockSpec((2, _BR, 128), lambda i: (0, i, 0)),
        out_shape=jax.ShapeDtypeStruct((2, NPAD, 128), jnp.float32),
    )(acc, cnt, x, wl, bl, wr)


def _tc_layer2(acc, cnt, h1, wl, bl, wr, w3l):
    def body(acc_ref, cnt_ref, h1_ref, wl_ref, bl_ref, wr_ref, w3_ref,
             h2_ref, y3_ref):
        agg = jnp.concatenate([acc_ref[0], acc_ref[1]], axis=1)
        agg = agg / jnp.maximum(cnt_ref[:, 0:1], 1.0)
        h1f = jnp.concatenate([h1_ref[0], h1_ref[1]], axis=1)
        h = _DOT(agg, wl_ref[...]) + bl_ref[...] + _DOT(h1f, wr_ref[...])
        h = jnp.maximum(h, 0.0)
        h2_ref[...] = h
        y3_ref[0] = _DOT(h, w3_ref[0])
        y3_ref[1] = _DOT(h, w3_ref[1])

    return pl.pallas_call(
        body,
        grid=(NPAD // _BR,),
        in_specs=[
            pl.BlockSpec((2, _BR, 128), lambda i: (0, i, 0)),
            pl.BlockSpec((_BR, 8), lambda i: (i, 0)),
            pl.BlockSpec((2, _BR, 128), lambda i: (0, i, 0)),
            pl.BlockSpec((512, 256), lambda i: (0, 0)),
            pl.BlockSpec((1, 512), lambda i: (0, 0)),
            pl.BlockSpec((512, 256), lambda i: (0, 0)),
            pl.BlockSpec((2, 64, 512), lambda i: (0, 0, 0)),
        ],
        out_specs=[
            pl.BlockSpec((_BR, 512), lambda i: (i, 0)),
            pl.BlockSpec((2, _BR, 64), lambda i: (0, i, 0)),
        ],
        out_shape=[
            jax.ShapeDtypeStruct((NPAD, 512), jnp.float32),
            jax.ShapeDtypeStruct((2, NPAD, 64), jnp.float32),
        ],
    )(acc, cnt, h1, wl, bl, wr, w3l)


def _tc_layer3(acc, cnt, h2, wr, bl):
    def body(acc_ref, cnt_ref, h2_ref, wr_ref, bl_ref, out_ref):
        agg = jnp.concatenate([acc_ref[0], acc_ref[1]], axis=1)
        agg = agg / jnp.maximum(cnt_ref[:, 0:1], 1.0)
        out_ref[...] = agg + bl_ref[...] + _DOT(h2_ref[...], wr_ref[...])

    return pl.pallas_call(
        body,
        grid=(NPAD // _BR,),
        in_specs=[
            pl.BlockSpec((2, _BR, 64), lambda i: (0, i, 0)),
            pl.BlockSpec((_BR, 8), lambda i: (i, 0)),
            pl.BlockSpec((_BR, 512), lambda i: (i, 0)),
            pl.BlockSpec((128, 512), lambda i: (0, 0)),
            pl.BlockSpec((1, 128), lambda i: (0, 0)),
        ],
        out_specs=pl.BlockSpec((_BR, 128), lambda i: (i, 0)),
        out_shape=jax.ShapeDtypeStruct((NPAD, 128), jnp.float32),
    )(acc, cnt, h2, wr, bl)


def kernel(x, edge_index, W1_l, b1_l, W1_r, W2_l, b2_l, W2_r, W3_l, b3_l, W3_r):
    x = x.astype(jnp.float32)
    src = edge_index[0].astype(jnp.int32)
    dst = edge_index[1].astype(jnp.int32)

    npad_e = EPAD - E
    pad_ar = jnp.arange(npad_e, dtype=jnp.int32)
    src_pad = jnp.concatenate([src, pad_ar % N])
    dst_pad = jnp.concatenate([dst, N + pad_ar % (NPAD - N)])
    src_t = src_pad.reshape(NTILE, NCHUNK, CHUNK)
    src2 = jnp.stack([src_t, src_t + NPAD])
    dst_t = dst_pad.reshape(NTILE, NCHUNK, CHUNK)

    zeros64 = jnp.zeros((NPAD, 64), jnp.float32)
    zeros128 = jnp.zeros((NPAD, 128), jnp.float32)
    zeros8 = jnp.zeros((NPAD, 8), jnp.float32)
    ones8 = jnp.ones((CHUNK, 8), jnp.float32)

    xpad = jnp.zeros((NPAD, D_IN), jnp.float32).at[:N].set(x)
    t1 = jnp.concatenate([xpad[:, :64], xpad[:, 64:]], axis=0)

    acc1, cnt = _make_agg(64, True)(t1, src2, dst_t, zeros64, ones8, zeros8)
    h1 = _tc_layer1(acc1.reshape(2, NPAD, 64), cnt, xpad,
                    W1_l, b1_l.reshape(1, -1), W1_r)

    acc2 = _make_agg(128, False)(h1.reshape(2 * NPAD, 128), src2, dst_t,
                                 zeros128)
    h2, y3 = _tc_layer2(acc2.reshape(2, NPAD, 128), cnt, h1,
                        W2_l, b2_l.reshape(1, -1), W2_r,
                        W3_l.reshape(2, 64, 512))

    acc3 = _make_agg(64, False)(y3.reshape(2 * NPAD, 64), src2, dst_t,
                                zeros64)
    out = _tc_layer3(acc3.reshape(2, NPAD, 64), cnt, h2,
                     W3_r, b3_l.reshape(1, -1))
    return out[:N]


# speedup vs baseline: 11.6235x; 1.0940x over previous
import functools

import jax
import jax.numpy as jnp
from jax import lax
from jax.experimental import pallas as pl
from jax.experimental.pallas import tpu as pltpu
from jax.experimental.pallas import tpu_sc as plsc

N = 10000
E = 320000
NPAD = 10240
D_IN = 128

NTILE = 16
NCORE = 2
CHUNK = 128
NCHUNK = 160
GRP = 8
NGRP = NCHUNK // GRP
EPT = CHUNK * NCHUNK
EPAD = EPT * NTILE
ROWS_PT = NPAD // NTILE


def _make_agg(W, with_counts):
    mesh = plsc.VectorSubcoreMesh(core_axis_name="c", subcore_axis_name="s")
    nbuf = 2 if W == 128 else 4
    npre = nbuf - 1
    out_type = [jax.ShapeDtypeStruct((2 * NPAD, W), jnp.float32)]
    scratch = [
        pltpu.VMEM_SHARED((NPAD, W), jnp.float32),
        pltpu.VMEM((2, GRP, CHUNK), jnp.int32),
        pltpu.VMEM((2, GRP, CHUNK), jnp.int32),
        [pltpu.VMEM((CHUNK, W), jnp.float32) for _ in range(nbuf)],
        [pltpu.SemaphoreType.DMA for _ in range(nbuf)],
        pltpu.SemaphoreType.DMA,
    ]
    if with_counts:
        out_type.append(jax.ShapeDtypeStruct((NPAD, 8), jnp.float32))
        scratch += [
            pltpu.VMEM_SHARED((NPAD, 8), jnp.float32),
            pltpu.VMEM((CHUNK, 8), jnp.float32),
        ]

    def body_common(table, src2, dstt, zeros_w, out, acc, sidx, didx, bufs,
                    sems, isem, c, t, cnt_stuff):
        r0 = t * ROWS_PT
        pltpu.sync_copy(zeros_w.at[pl.ds(r0, ROWS_PT)],
                        acc.at[pl.ds(r0, ROWS_PT)])
        if cnt_stuff is not None:
            ones8, zeros8, cnt_out, cntacc, onesb = cnt_stuff
            pltpu.sync_copy(zeros8.at[pl.ds(r0, ROWS_PT)],
                            cntacc.at[pl.ds(r0, ROWS_PT)])
            pltpu.sync_copy(ones8, onesb)

        def gather_start(slot, j, b):
            pltpu.async_copy(table.at[sidx.at[slot, j]], bufs[b], sems[b])

        def gather_wait(b):
            pltpu.make_async_copy(table.at[sidx.at[0, 0]], bufs[b],
                                  sems[b]).wait()

        def idx_start(g):
            off = jnp.minimum(g, NGRP - 1) * GRP
            slot = lax.rem(g, 2)
            pltpu.async_copy(src2.at[c, t, pl.ds(off, GRP)],
                             sidx.at[slot], isem)
            pltpu.async_copy(dstt.at[t, pl.ds(off, GRP)],
                             didx.at[slot], isem)

        def idx_wait(g):
            slot = lax.rem(g, 2)
            pltpu.make_async_copy(src2.at[c, t, pl.ds(0, GRP)],
                                  sidx.at[slot], isem).wait()
            pltpu.make_async_copy(dstt.at[t, pl.ds(0, GRP)],
                                  didx.at[slot], isem).wait()

        idx_start(0)
        idx_wait(0)
        for j in range(npre):
            gather_start(0, j, j)
        plsc.subcore_barrier()

        def group(g, carry):
            slot = lax.rem(g, 2)
            idx_start(g + 1)
            for j in range(GRP):
                b = j % nbuf
                gather_wait(b)
                nj = j + npre
                if nj < GRP:
                    gather_start(slot, nj, nj % nbuf)
                pltpu.sync_copy(bufs[b], acc.at[didx.at[slot, j]], add=True)
                if cnt_stuff is not None:
                    pltpu.sync_copy(onesb, cntacc.at[didx.at[slot, j]],
                                    add=True)
            idx_wait(g + 1)
            for j in range(npre):
                gather_start(lax.rem(g + 1, 2), j, j % nbuf)
            return carry

        lax.fori_loop(0, NGRP, group, 0)
        for j in range(npre):
            gather_wait(j % nbuf)
        plsc.subcore_barrier()
        pltpu.sync_copy(acc.at[pl.ds(r0, ROWS_PT)],
                        out.at[pl.ds(c * NPAD + r0, ROWS_PT)])
        if cnt_stuff is not None:
            @pl.when(c == 0)
            def _():
                pltpu.sync_copy(cntacc.at[pl.ds(r0, ROWS_PT)],
                                cnt_out.at[pl.ds(r0, ROWS_PT)])

    if with_counts:
        def body(table, src2, dstt, zeros_w, ones8, zeros8, out, cnt_out,
                 acc, sidx, didx, bufs, sems, isem, cntacc, onesb):
            c = lax.axis_index("c")
            t = lax.axis_index("s")
            body_common(table, src2, dstt, zeros_w, out, acc, sidx, didx,
                        bufs, sems, isem, c, t,
                        (ones8, zeros8, cnt_out, cntacc, onesb))
    else:
        def body(table, src2, dstt, zeros_w, out,
                 acc, sidx, didx, bufs, sems, isem):
            c = lax.axis_index("c")
            t = lax.axis_index("s")
            body_common(table, src2, dstt, zeros_w, out, acc, sidx, didx,
                        bufs, sems, isem, c, t, None)

    out_type = tuple(out_type) if with_counts else out_type[0]
    return pl.kernel(body, out_type=out_type, mesh=mesh,
                     scratch_types=scratch,
                     compiler_params=pltpu.CompilerParams(
                         use_tc_tiling_on_sc=False))


_DOT = functools.partial(
    lax.dot_general,
    dimension_numbers=(((1,), (1,)), ((), ())),
    preferred_element_type=jnp.float32,
)

_BR = 256


def _tc_rpath(h, wr, bl, d_in, d_out):
    def body(h_ref, wr_ref, bl_ref, xr_ref):
        xr_ref[...] = _DOT(h_ref[...], wr_ref[...]) + bl_ref[...]

    return pl.pallas_call(
        body,
        grid=(NPAD // _BR,),
        in_specs=[
            pl.BlockSpec((_BR, d_in), lambda i: (i, 0)),
            pl.BlockSpec((d_out, d_in), lambda i: (0, 0)),
            pl.BlockSpec((1, d_out), lambda i: (0, 0)),
        ],
        out_specs=pl.BlockSpec((_BR, d_out), lambda i: (i, 0)),
        out_shape=jax.ShapeDtypeStruct((NPAD, d_out), jnp.float32),
    )(h, wr, bl)


def _tc_rpath2(h1, wra, wrb, bl):
    def body(h1_ref, wra_ref, wrb_ref, bl_ref, xr_ref):
        xr_ref[...] = (_DOT(h1_ref[0], wra_ref[...])
                       + _DOT(h1_ref[1], wrb_ref[...]) + bl_ref[...])

    return pl.pallas_call(
        body,
        grid=(NPAD // _BR,),
        in_specs=[
            pl.BlockSpec((2, _BR, 128), lambda i: (0, i, 0)),
            pl.BlockSpec((512, 128), lambda i: (0, 0)),
            pl.BlockSpec((512, 128), lambda i: (0, 0)),
            pl.BlockSpec((1, 512), lambda i: (0, 0)),
        ],
        out_specs=pl.BlockSpec((_BR, 512), lambda i: (i, 0)),
        out_shape=jax.ShapeDtypeStruct((NPAD, 512), jnp.float32),
    )(h1, wra, wrb, bl)


def _tc_layer1(acc, cnt, xr, wl):
    def body(acc_ref, cnt_ref, xr_ref, wl_ref, h1_ref):
        agg = jnp.concatenate([acc_ref[0], acc_ref[1]], axis=1)
        agg = agg / jnp.maximum(cnt_ref[:, 0:1], 1.0)
        h = _DOT(agg, wl_ref[...]) + xr_ref[...]
        h = jnp.maximum(h, 0.0)
        h1_ref[0] = h[:, :128]
        h1_ref[1] = h[:, 128:]

    return pl.pallas_call(
        body,
        grid=(NPAD // _BR,),
        in_specs=[
            pl.BlockSpec((2, _BR, 64), lambda i: (0, i, 0)),
            pl.BlockSpec((_BR, 8), lambda i: (i, 0)),
            pl.BlockSpec((_BR, 256), lambda i: (i, 0)),
            pl.BlockSpec((256, 128), lambda i: (0, 0)),
        ],
        out_specs=pl.BlockSpec((2, _BR, 128), lambda i: (0, i, 0)),
        out_shape=jax.ShapeDtypeStruct((2, NPAD, 128), jnp.float32),
    )(acc, cnt, xr, wl)


def _tc_layer2(acc, cnt, xr, wl, w3l):
    def body(acc_ref, cnt_ref, xr_ref, wl_ref, w3_ref, h2_ref, y3_ref):
        agg = jnp.concatenate([acc_ref[0], acc_ref[1]], axis=1)
        agg = agg / jnp.maximum(cnt_ref[:, 0:1], 1.0)
        h = _DOT(agg, wl_ref[...]) + xr_ref[...]
        h = jnp.maximum(h, 0.0)
        h2_ref[...] = h
        y3_ref[0] = _DOT(h, w3_ref[0])
        y3_ref[1] = _DOT(h, w3_ref[1])

    return pl.pallas_call(
        body,
        grid=(NPAD // _BR,),
        in_specs=[
            pl.BlockSpec((2, _BR, 128), lambda i: (0, i, 0)),
            pl.BlockSpec((_BR, 8), lambda i: (i, 0)),
            pl.BlockSpec((_BR, 512), lambda i: (i, 0)),
            pl.BlockSpec((512, 256), lambda i: (0, 0)),
            pl.BlockSpec((2, 64, 512), lambda i: (0, 0, 0)),
        ],
        out_specs=[
            pl.BlockSpec((_BR, 512), lambda i: (i, 0)),
            pl.BlockSpec((2, _BR, 64), lambda i: (0, i, 0)),
        ],
        out_shape=[
            jax.ShapeDtypeStruct((NPAD, 512), jnp.float32),
            jax.ShapeDtypeStruct((2, NPAD, 64), jnp.float32),
        ],
    )(acc, cnt, xr, wl, w3l)


def _tc_layer3(acc, cnt, xr):
    def body(acc_ref, cnt_ref, xr_ref, out_ref):
        agg = jnp.concatenate([acc_ref[0], acc_ref[1]], axis=1)
        agg = agg / jnp.maximum(cnt_ref[:, 0:1], 1.0)
        out_ref[...] = agg + xr_ref[...]

    return pl.pallas_call(
        body,
        grid=(NPAD // _BR,),
        in_specs=[
            pl.BlockSpec((2, _BR, 64), lambda i: (0, i, 0)),
            pl.BlockSpec((_BR, 8), lambda i: (i, 0)),
            pl.BlockSpec((_BR, 128), lambda i: (i, 0)),
        ],
        out_specs=pl.BlockSpec((_BR, 128), lambda i: (i, 0)),
        out_shape=jax.ShapeDtypeStruct((NPAD, 128), jnp.float32),
    )(acc, cnt, xr)


def kernel(x, edge_index, W1_l, b1_l, W1_r, W2_l, b2_l, W2_r, W3_l, b3_l, W3_r):
    x = x.astype(jnp.float32)
    src = edge_index[0].astype(jnp.int32)
    dst = edge_index[1].astype(jnp.int32)

    npad_e = EPAD - E
    pad_ar = jnp.arange(npad_e, dtype=jnp.int32)
    src_pad = jnp.concatenate([src, pad_ar % N])
    dst_pad = jnp.concatenate([dst, N + pad_ar % (NPAD - N)])
    src_t = src_pad.reshape(NTILE, NCHUNK, CHUNK)
    src2 = jnp.stack([src_t, src_t + NPAD])
    dst_t = dst_pad.reshape(NTILE, NCHUNK, CHUNK)

    zeros64 = jnp.zeros((NPAD, 64), jnp.float32)
    zeros128 = jnp.zeros((NPAD, 128), jnp.float32)
    zeros8 = jnp.zeros((NPAD, 8), jnp.float32)
    ones8 = jnp.ones((CHUNK, 8), jnp.float32)

    xpad = jnp.zeros((NPAD, D_IN), jnp.float32).at[:N].set(x)
    t1 = jnp.concatenate([xpad[:, :64], xpad[:, 64:]], axis=0)

    acc1, cnt = _make_agg(64, True)(t1, src2, dst_t, zeros64, ones8, zeros8)
    xr1 = _tc_rpath(xpad, W1_r, b1_l.reshape(1, -1), 128, 256)
    h1 = _tc_layer1(acc1.reshape(2, NPAD, 64), cnt, xr1, W1_l)

    acc2 = _make_agg(128, False)(h1.reshape(2 * NPAD, 128), src2, dst_t,
                                 zeros128)
    xr2 = _tc_rpath2(h1, W2_r[:, :128], W2_r[:, 128:], b2_l.reshape(1, -1))
    h2, y3 = _tc_layer2(acc2.reshape(2, NPAD, 128), cnt, xr2,
                        W2_l, W3_l.reshape(2, 64, 512))

    acc3 = _make_agg(64, False)(y3.reshape(2 * NPAD, 64), src2, dst_t,
                                zeros64)
    xr3 = _tc_rpath(h2, W3_r, b3_l.reshape(1, -1), 512, 128)
    out = _tc_layer3(acc3.reshape(2, NPAD, 64), cnt, xr3)
    return out[:N]
